# baseline (device time: 27260 ns/iter reference)
import jax
import jax.numpy as jnp
from jax import lax
from jax.experimental import pallas as pl
from jax.experimental.pallas import tpu as pltpu

T = 512
D = 1024
V_SHARD = 8192
VC = 1024
NC = V_SHARD // VC


def kernel(x, W, labels):
    def body(
        x_ref,
        w_ref,
        lab_ref,
        out_ref,
        acc_s_ref,
        acc_l_ref,
        payload_ref,
        recv_ref,
        send_sem,
        recv_sem,
    ):
        i = pl.program_id(0)
        my_x = lax.axis_index("x")
        my_y = lax.axis_index("y")
        my_z = lax.axis_index("z")

        logits = jnp.dot(
            x_ref[...].astype(jnp.bfloat16),
            w_ref[...].astype(jnp.bfloat16),
            preferred_element_type=jnp.float32,
        )
        e = jnp.exp(logits.astype(jnp.bfloat16))

        @pl.when(i == 0)
        def _():
            barrier_sem = pltpu.get_barrier_semaphore()
            pl.semaphore_signal(
                barrier_sem,
                inc=1,
                device_id=(my_x, my_y, 1 - my_z),
                device_id_type=pl.DeviceIdType.MESH,
            )
            pl.semaphore_wait(barrier_sem, 1)

        local_lab = lab_ref[...] - my_z * V_SHARD - i * VC
        col = lax.broadcasted_iota(jnp.int32, (T, VC), 1)
        masked = jnp.where(col == local_lab[:, None], logits, 0.0)

        ones = jnp.ones((VC, 128), jnp.float32)
        ones_bf = jnp.ones((VC, 128), jnp.bfloat16)
        s_part = jnp.dot(e, ones_bf, preferred_element_type=jnp.float32)
        l_part = jnp.dot(masked, ones, preferred_element_type=jnp.float32)

        @pl.when(i == 0)
        def _():
            acc_s_ref[...] = s_part
            acc_l_ref[...] = l_part

        @pl.when(i > 0)
        def _():
            acc_s_ref[...] += s_part
            acc_l_ref[...] += l_part

        @pl.when(i == NC - 1)
        def _():
            payload_ref[0, :] = acc_s_ref[:, 0]
            payload_ref[1, :] = acc_l_ref[:, 0]

            rdma = pltpu.make_async_remote_copy(
                src_ref=payload_ref,
                dst_ref=recv_ref,
                send_sem=send_sem,
                recv_sem=recv_sem,
                device_id=(my_x, my_y, 1 - my_z),
                device_id_type=pl.DeviceIdType.MESH,
            )
            rdma.start()
            rdma.wait()

            s_tot = payload_ref[0, :] + recv_ref[0, :]
            lab_tot = payload_ref[1, :] + recv_ref[1, :]
            out_ref[...] = jnp.log(s_tot) - lab_tot

    return pl.pallas_call(
        body,
        grid=(NC,),
        out_shape=jax.ShapeDtypeStruct((T,), jnp.float32),
        in_specs=[
            pl.BlockSpec((T, D), lambda i: (0, 0)),
            pl.BlockSpec((D, VC), lambda i: (0, i)),
            pl.BlockSpec((T,), lambda i: (0,)),
        ],
        out_specs=pl.BlockSpec((T,), lambda i: (0,)),
        scratch_shapes=[
            pltpu.VMEM((T, 128), jnp.float32),
            pltpu.VMEM((T, 128), jnp.float32),
            pltpu.VMEM((2, T), jnp.float32),
            pltpu.VMEM((2, T), jnp.float32),
            pltpu.SemaphoreType.DMA,
            pltpu.SemaphoreType.DMA,
        ],
        compiler_params=pltpu.CompilerParams(
            vmem_limit_bytes=60 * 1024 * 1024,
            collective_id=0,
        ),
    )(x, W, labels)


# device time: 26299 ns/iter; 1.0365x vs baseline; 1.0365x over previous
import jax
import jax.numpy as jnp
from jax import lax
from jax.experimental import pallas as pl
from jax.experimental.pallas import tpu as pltpu

T = 512
D = 1024
V_SHARD = 8192
VC = 2048
NC = V_SHARD // VC


def kernel(x, W, labels):
    def body(
        x_ref,
        w_ref,
        lab_ref,
        out_ref,
        acc_s_ref,
        acc_l_ref,
        payload_ref,
        recv_ref,
        send_sem,
        recv_sem,
    ):
        i = pl.program_id(0)
        my_x = lax.axis_index("x")
        my_y = lax.axis_index("y")
        my_z = lax.axis_index("z")

        logits = jnp.dot(
            x_ref[...].astype(jnp.bfloat16),
            w_ref[...].astype(jnp.bfloat16),
            preferred_element_type=jnp.float32,
        )
        e = jnp.exp(logits.astype(jnp.bfloat16))

        @pl.when(i == 0)
        def _():
            barrier_sem = pltpu.get_barrier_semaphore()
            pl.semaphore_signal(
                barrier_sem,
                inc=1,
                device_id=(my_x, my_y, 1 - my_z),
                device_id_type=pl.DeviceIdType.MESH,
            )
            pl.semaphore_wait(barrier_sem, 1)

        local_lab = lab_ref[...] - my_z * V_SHARD - i * VC
        col = lax.broadcasted_iota(jnp.int32, (T, VC), 1)
        masked = jnp.where(col == local_lab[:, None], logits, 0.0)

        ones = jnp.ones((VC, 128), jnp.float32)
        ones_bf = jnp.ones((VC, 128), jnp.bfloat16)
        s_part = jnp.dot(e, ones_bf, preferred_element_type=jnp.float32)
        l_part = jnp.dot(masked, ones, preferred_element_type=jnp.float32)

        @pl.when(i == 0)
        def _():
            acc_s_ref[...] = s_part
            acc_l_ref[...] = l_part

        @pl.when(i > 0)
        def _():
            acc_s_ref[...] += s_part
            acc_l_ref[...] += l_part

        @pl.when(i == NC - 1)
        def _():
            payload_ref[0, :] = acc_s_ref[:, 0]
            payload_ref[1, :] = acc_l_ref[:, 0]

            rdma = pltpu.make_async_remote_copy(
                src_ref=payload_ref,
                dst_ref=recv_ref,
                send_sem=send_sem,
                recv_sem=recv_sem,
                device_id=(my_x, my_y, 1 - my_z),
                device_id_type=pl.DeviceIdType.MESH,
            )
            rdma.start()
            rdma.wait()

            s_tot = payload_ref[0, :] + recv_ref[0, :]
            lab_tot = payload_ref[1, :] + recv_ref[1, :]
            out_ref[...] = jnp.log(s_tot) - lab_tot

    return pl.pallas_call(
        body,
        grid=(NC,),
        out_shape=jax.ShapeDtypeStruct((T,), jnp.float32),
        in_specs=[
            pl.BlockSpec((T, D), lambda i: (0, 0)),
            pl.BlockSpec((D, VC), lambda i: (0, i)),
            pl.BlockSpec((T,), lambda i: (0,)),
        ],
        out_specs=pl.BlockSpec((T,), lambda i: (0,)),
        scratch_shapes=[
            pltpu.VMEM((T, 128), jnp.float32),
            pltpu.VMEM((T, 128), jnp.float32),
            pltpu.VMEM((2, T), jnp.float32),
            pltpu.VMEM((2, T), jnp.float32),
            pltpu.SemaphoreType.DMA,
            pltpu.SemaphoreType.DMA,
        ],
        compiler_params=pltpu.CompilerParams(
            vmem_limit_bytes=60 * 1024 * 1024,
            collective_id=0,
        ),
    )(x, W, labels)
